# Initial kernel scaffold; baseline (speedup 1.0000x reference)
import jax
import jax.numpy as jnp
from jax.experimental import pallas as pl


def kernel(x, edge_index, edge_attr, ee1, ee2, W1, b1, W2, b2):
    raise NotImplementedError("write your pallas kernel here")


# speedup vs baseline: 1.7067x; 1.7067x over previous
import jax
import jax.numpy as jnp
from jax import lax
from jax.experimental import pallas as pl
from jax.experimental.pallas import tpu as pltpu

EB = 512
KPAD = 32


def _edge_accumulate(x, src2d, dst2d, k2d, ee):
    n, dfeat = x.shape
    nblk, _, eb = src2d.shape

    def body(src_ref, dst_ref, k_ref, x_ref, ee_ref, o_ref, acc_ref):
        step = pl.program_id(0)

        @pl.when(step == 0)
        def _():
            acc_ref[...] = jnp.zeros_like(acc_ref)

        def edge(e, _):
            s0 = src_ref[0, 0, e]
            d0 = dst_ref[0, 0, e]
            k0 = k_ref[0, 0, e]
            row = x_ref[pl.ds(s0, 1), :] + ee_ref[pl.ds(k0, 1), :]
            acc_ref[pl.ds(d0, 1), :] += row
            return 0

        lax.fori_loop(0, eb, edge, 0)

        @pl.when(step == nblk - 1)
        def _():
            o_ref[...] = acc_ref[...]

    return pl.pallas_call(
        body,
        grid=(nblk,),
        in_specs=[
            pl.BlockSpec((1, 1, eb), lambda i: (i, 0, 0), memory_space=pltpu.SMEM),
            pl.BlockSpec((1, 1, eb), lambda i: (i, 0, 0), memory_space=pltpu.SMEM),
            pl.BlockSpec((1, 1, eb), lambda i: (i, 0, 0), memory_space=pltpu.SMEM),
            pl.BlockSpec((n, dfeat), lambda i: (0, 0)),
            pl.BlockSpec((KPAD, dfeat), lambda i: (0, 0)),
        ],
        out_specs=pl.BlockSpec((n, dfeat), lambda i: (0, 0)),
        out_shape=jax.ShapeDtypeStruct((n, dfeat), jnp.float32),
        scratch_shapes=[pltpu.VMEM((n, dfeat), jnp.float32)],
    )(src2d, dst2d, k2d, x, ee)


def _combine_mlp(acc, x, eself, W1, b1, W2, b2):
    n, dfeat = x.shape
    dh = W1.shape[1]
    blk = 1000
    assert n % blk == 0

    def body(a_ref, x_ref, es_ref, w1_ref, b1_ref, w2_ref, b2_ref, o_ref):
        agg = a_ref[...] + x_ref[...] + es_ref[...]
        h = jnp.dot(agg, w1_ref[...], preferred_element_type=jnp.float32) + b1_ref[...]
        h = jnp.maximum(h, 0.0)
        o_ref[...] = jnp.dot(h, w2_ref[...], preferred_element_type=jnp.float32) + b2_ref[...]

    return pl.pallas_call(
        body,
        grid=(n // blk,),
        in_specs=[
            pl.BlockSpec((blk, dfeat), lambda i: (i, 0)),
            pl.BlockSpec((blk, dfeat), lambda i: (i, 0)),
            pl.BlockSpec((1, dfeat), lambda i: (0, 0)),
            pl.BlockSpec((dfeat, dh), lambda i: (0, 0)),
            pl.BlockSpec((1, dh), lambda i: (0, 0)),
            pl.BlockSpec((dh, dfeat), lambda i: (0, 0)),
            pl.BlockSpec((1, dfeat), lambda i: (0, 0)),
        ],
        out_specs=pl.BlockSpec((blk, dfeat), lambda i: (i, 0)),
        out_shape=jax.ShapeDtypeStruct((n, dfeat), jnp.float32),
    )(acc, x, eself, W1, b1, W2, b2)


def kernel(x, edge_index, edge_attr, ee1, ee2, W1, b1, W2, b2):
    n, dfeat = x.shape
    e = edge_index.shape[1]
    assert e % EB == 0
    nblk = e // EB

    src2d = edge_index[0].reshape(nblk, 1, EB)
    dst2d = edge_index[1].reshape(nblk, 1, EB)
    k2d = (edge_attr[:, 0] * 3 + edge_attr[:, 1]).astype(jnp.int32).reshape(nblk, 1, EB)

    kk = jnp.arange(KPAD)
    nt, ndir = ee1.shape[0], ee2.shape[0]
    valid = kk < nt * ndir
    ee = jnp.where(
        valid[:, None],
        ee1[jnp.clip(kk // 3, 0, nt - 1)] + ee2[jnp.clip(kk % 3, 0, ndir - 1)],
        0.0,
    ).astype(jnp.float32)
    eself = (ee1[4] + ee2[0]).reshape(1, dfeat).astype(jnp.float32)

    acc = _edge_accumulate(x, src2d, dst2d, k2d, ee)
    return _combine_mlp(acc, x, eself, W1, b1.reshape(1, -1), W2, b2.reshape(1, -1))


# speedup vs baseline: 3.5089x; 2.0559x over previous
import jax
import jax.numpy as jnp
from jax import lax
from jax.experimental import pallas as pl
from jax.experimental.pallas import tpu as pltpu

EB = 512
KPAD = 32


def _edge_accumulate(x, src2d, dst2d, k2d, ee):
    n, dfeat = x.shape
    nblk, _, eb = src2d.shape

    def body(src_ref, dst_ref, k_ref, x_ref, ee_ref, o_ref, acc_ref):
        step = pl.program_id(0)

        @pl.when(step == 0)
        def _():
            acc_ref[...] = jnp.zeros_like(acc_ref)

        def edge(e, _):
            s0 = src_ref[0, 0, e]
            d0 = dst_ref[0, 0, e]
            k0 = k_ref[0, 0, e]
            row = x_ref[pl.ds(s0, 1), :] + ee_ref[pl.ds(k0, 1), :]
            acc_ref[pl.ds(d0, 1), :] += row
            return 0

        lax.fori_loop(0, eb, edge, 0, unroll=8)

        @pl.when(step == nblk - 1)
        def _():
            o_ref[...] = acc_ref[...]

    return pl.pallas_call(
        body,
        grid=(nblk,),
        in_specs=[
            pl.BlockSpec((1, 1, eb), lambda i: (i, 0, 0), memory_space=pltpu.SMEM),
            pl.BlockSpec((1, 1, eb), lambda i: (i, 0, 0), memory_space=pltpu.SMEM),
            pl.BlockSpec((1, 1, eb), lambda i: (i, 0, 0), memory_space=pltpu.SMEM),
            pl.BlockSpec((n, dfeat), lambda i: (0, 0)),
            pl.BlockSpec((KPAD, dfeat), lambda i: (0, 0)),
        ],
        out_specs=pl.BlockSpec((n, dfeat), lambda i: (0, 0)),
        out_shape=jax.ShapeDtypeStruct((n, dfeat), jnp.float32),
        scratch_shapes=[pltpu.VMEM((n, dfeat), jnp.float32)],
    )(src2d, dst2d, k2d, x, ee)


def _combine_mlp(acc, x, eself, W1, b1, W2, b2):
    n, dfeat = x.shape
    dh = W1.shape[1]
    blk = 1000
    assert n % blk == 0

    def body(a_ref, x_ref, es_ref, w1_ref, b1_ref, w2_ref, b2_ref, o_ref):
        agg = a_ref[...] + x_ref[...] + es_ref[...]
        h = jnp.dot(agg, w1_ref[...], preferred_element_type=jnp.float32) + b1_ref[...]
        h = jnp.maximum(h, 0.0)
        o_ref[...] = jnp.dot(h, w2_ref[...], preferred_element_type=jnp.float32) + b2_ref[...]

    return pl.pallas_call(
        body,
        grid=(n // blk,),
        in_specs=[
            pl.BlockSpec((blk, dfeat), lambda i: (i, 0)),
            pl.BlockSpec((blk, dfeat), lambda i: (i, 0)),
            pl.BlockSpec((1, dfeat), lambda i: (0, 0)),
            pl.BlockSpec((dfeat, dh), lambda i: (0, 0)),
            pl.BlockSpec((1, dh), lambda i: (0, 0)),
            pl.BlockSpec((dh, dfeat), lambda i: (0, 0)),
            pl.BlockSpec((1, dfeat), lambda i: (0, 0)),
        ],
        out_specs=pl.BlockSpec((blk, dfeat), lambda i: (i, 0)),
        out_shape=jax.ShapeDtypeStruct((n, dfeat), jnp.float32),
    )(acc, x, eself, W1, b1, W2, b2)


def kernel(x, edge_index, edge_attr, ee1, ee2, W1, b1, W2, b2):
    n, dfeat = x.shape
    e = edge_index.shape[1]
    assert e % EB == 0
    nblk = e // EB

    src2d = edge_index[0].reshape(nblk, 1, EB)
    dst2d = edge_index[1].reshape(nblk, 1, EB)
    k2d = (edge_attr[:, 0] * 3 + edge_attr[:, 1]).astype(jnp.int32).reshape(nblk, 1, EB)

    kk = jnp.arange(KPAD)
    nt, ndir = ee1.shape[0], ee2.shape[0]
    valid = kk < nt * ndir
    ee = jnp.where(
        valid[:, None],
        ee1[jnp.clip(kk // 3, 0, nt - 1)] + ee2[jnp.clip(kk % 3, 0, ndir - 1)],
        0.0,
    ).astype(jnp.float32)
    eself = (ee1[4] + ee2[0]).reshape(1, dfeat).astype(jnp.float32)

    acc = _edge_accumulate(x, src2d, dst2d, k2d, ee)
    return _combine_mlp(acc, x, eself, W1, b1.reshape(1, -1), W2, b2.reshape(1, -1))
